# Initial kernel scaffold; baseline (speedup 1.0000x reference)
import jax
import jax.numpy as jnp
from jax.experimental import pallas as pl


def kernel(x, edge_index, eps, p):
    raise NotImplementedError("write your pallas kernel here")


# speedup vs baseline: 13.5242x; 13.5242x over previous
import jax
import jax.numpy as jnp
from jax import lax
from jax.experimental import pallas as pl
from jax.experimental.pallas import tpu as pltpu
from jax.experimental.pallas import tpu_sc as plsc

N = 10000
E = 320000
D = 128
CH = 80
NC = 2
NS = 16
NP = 10240
RPT = NP // NS
ZR = 128
IB = 25
K1_ROWS = E // CH // (NC * NS)
K3_ROWS = E // CH // NS
K1_G = K1_ROWS // IB
K3_G = K3_ROWS // IB
RB = 1000


def _deg_body(col_hbm, ones_hbm, z_hbm, dp_hbm, ones_v, idx_v, acc):
    c = lax.axis_index("c")
    s = lax.axis_index("s")
    wid = c * NS + s
    pltpu.sync_copy(ones_hbm, ones_v)
    for k in range(RPT // ZR):
        pltpu.sync_copy(z_hbm, acc.at[pl.ds(s * RPT + k * ZR, ZR)])
    plsc.subcore_barrier()

    def outer(g, carry):
        pltpu.sync_copy(col_hbm.at[wid, g], idx_v)

        def step(i, c2):
            pltpu.sync_copy(ones_v, acc.at[idx_v.at[i]], add=True)
            return c2

        return lax.fori_loop(0, IB, step, carry)

    lax.fori_loop(0, K1_G, outer, 0)
    plsc.subcore_barrier()
    sl = pl.ds(s * RPT, RPT)
    pltpu.sync_copy(acc.at[sl], dp_hbm.at[c, sl])


_deg_call = pl.kernel(
    _deg_body,
    out_type=jax.ShapeDtypeStruct((NC, NP, D), jnp.float32),
    mesh=plsc.VectorSubcoreMesh(core_axis_name="c", subcore_axis_name="s"),
    scratch_types=[
        pltpu.VMEM((CH, D), jnp.float32),
        pltpu.VMEM((IB, CH), jnp.int32),
        pltpu.VMEM_SHARED((NP, D), jnp.float32),
    ],
)


def _agg_body(row_hbm, col_hbm, u_hbm, v_hbm, z_hbm,
              U_hbm, V_hbm, idxr, idxc, gbuf, acc, sem):
    c = lax.axis_index("c")
    s = lax.axis_index("s")
    for k in range(RPT // ZR):
        pltpu.sync_copy(z_hbm, acc.at[pl.ds(s * RPT + k * ZR, ZR)])
    plsc.subcore_barrier()

    def outer(g, carry):
        pltpu.sync_copy(row_hbm.at[s, g], idxr)
        pltpu.sync_copy(col_hbm.at[s, g], idxc)

        def step(i, c2):
            @pl.when(c == 0)
            def _():
                pltpu.async_copy(u_hbm.at[idxc.at[i]], gbuf, sem).wait()

            @pl.when(c == 1)
            def _():
                pltpu.async_copy(v_hbm.at[idxc.at[i]], gbuf, sem).wait()

            pltpu.sync_copy(gbuf, acc.at[idxr.at[i]], add=True)
            return c2

        return lax.fori_loop(0, IB, step, carry)

    lax.fori_loop(0, K3_G, outer, 0)
    plsc.subcore_barrier()
    sl = pl.ds(s * RPT, RPT)

    @pl.when(c == 0)
    def _():
        pltpu.sync_copy(acc.at[sl], U_hbm.at[sl])

    @pl.when(c == 1)
    def _():
        pltpu.sync_copy(acc.at[sl], V_hbm.at[sl])


_agg_call = pl.kernel(
    _agg_body,
    out_type=(
        jax.ShapeDtypeStruct((NP, D), jnp.float32),
        jax.ShapeDtypeStruct((NP, D), jnp.float32),
    ),
    mesh=plsc.VectorSubcoreMesh(core_axis_name="c", subcore_axis_name="s"),
    scratch_types=[
        pltpu.VMEM((IB, CH), jnp.int32),
        pltpu.VMEM((IB, CH), jnp.int32),
        pltpu.VMEM((CH, D), jnp.float32),
        pltpu.VMEM_SHARED((NP, D), jnp.float32),
        pltpu.SemaphoreType.DMA,
    ],
)


def _mx_body(x_ref, o_ref):
    o_ref[0, 0] = jnp.max(x_ref[...])


def _mx_call(x):
    return pl.pallas_call(
        _mx_body,
        out_shape=jax.ShapeDtypeStruct((1, 1), jnp.float32),
        out_specs=pl.BlockSpec(memory_space=pltpu.SMEM),
    )(x)


def _tab_body(mx_ref, p_ref, x_ref, dp_ref, u_ref, v_ref):
    pp = jax.nn.sigmoid(p_ref[0, 0]) * 2.0
    m = pp * mx_ref[0, 0]
    dp = dp_ref[...]
    deg = dp[0, :, 0] + dp[1, :, 0]
    dis = jnp.where(deg > 0, lax.rsqrt(deg), 0.0)
    xb = x_ref[...]
    u = jnp.exp(pp * xb - m) * dis[:, None]
    u_ref[...] = u
    v_ref[...] = u * xb


def _tab_call(mx, p2, x, dp):
    return pl.pallas_call(
        _tab_body,
        grid=(N // RB,),
        in_specs=[
            pl.BlockSpec((1, 1), lambda i: (0, 0), memory_space=pltpu.SMEM),
            pl.BlockSpec((1, 1), lambda i: (0, 0), memory_space=pltpu.SMEM),
            pl.BlockSpec((RB, D), lambda i: (i, 0)),
            pl.BlockSpec((NC, RB, D), lambda i: (0, i, 0)),
        ],
        out_specs=[
            pl.BlockSpec((RB, D), lambda i: (i, 0)),
            pl.BlockSpec((RB, D), lambda i: (i, 0)),
        ],
        out_shape=[
            jax.ShapeDtypeStruct((N, D), jnp.float32),
            jax.ShapeDtypeStruct((N, D), jnp.float32),
        ],
    )(mx, p2, x, dp)


def _out_body(e_ref, U_ref, V_ref, x_ref, dp_ref, o_ref):
    dp = dp_ref[...]
    deg = dp[0, :, 0] + dp[1, :, 0]
    dis = jnp.where(deg > 0, lax.rsqrt(deg), 0.0)[:, None]
    xb = x_ref[...]
    agg = (dis * V_ref[...]) / (dis * U_ref[...] + 1e-6)
    o_ref[...] = agg + (1.0 + e_ref[0, 0]) * xb


def _out_call(e2, U, V, x, dp):
    return pl.pallas_call(
        _out_body,
        grid=(N // RB,),
        in_specs=[
            pl.BlockSpec((1, 1), lambda i: (0, 0), memory_space=pltpu.SMEM),
            pl.BlockSpec((RB, D), lambda i: (i, 0)),
            pl.BlockSpec((RB, D), lambda i: (i, 0)),
            pl.BlockSpec((RB, D), lambda i: (i, 0)),
            pl.BlockSpec((NC, RB, D), lambda i: (0, i, 0)),
        ],
        out_specs=pl.BlockSpec((RB, D), lambda i: (i, 0)),
        out_shape=jax.ShapeDtypeStruct((N, D), jnp.float32),
    )(e2, U, V, x, dp)


def kernel(x, edge_index, eps, p):
    row = edge_index[0]
    col = edge_index[1]
    row3 = row.reshape(NS, K3_G, IB, CH)
    col3 = col.reshape(NS, K3_G, IB, CH)
    col3d = col.reshape(NC * NS, K1_G, IB, CH)
    onesD = jnp.ones((CH, D), jnp.float32)
    zD = jnp.zeros((ZR, D), jnp.float32)

    dp = _deg_call(col3d, onesD, zD)
    mx = _mx_call(x)
    u, v = _tab_call(mx, p.reshape(1, 1), x, dp)
    U, V = _agg_call(row3, col3, u, v, zD)
    return _out_call(eps.reshape(1, 1), U, V, x, dp)


# speedup vs baseline: 16.4304x; 1.2149x over previous
import jax
import jax.numpy as jnp
from jax import lax
from jax.experimental import pallas as pl
from jax.experimental.pallas import tpu as pltpu
from jax.experimental.pallas import tpu_sc as plsc

N = 10000
E = 320000
D = 128
CH = 80
NC = 2
NS = 16
NP = 10240
RPT = NP // NS
ZR = 128
IB = 25
IBA = 50
GB = 2
K1_ROWS = E // CH // (NC * NS)
K3_ROWS = E // CH // NS
K1_G = K1_ROWS // IB
K3_G = K3_ROWS // IBA
RB = 1000


def _deg_body(col_hbm, ones_hbm, z_hbm, dp_hbm, ones_v, idx_v, acc):
    c = lax.axis_index("c")
    s = lax.axis_index("s")
    wid = c * NS + s
    pltpu.sync_copy(ones_hbm, ones_v)
    for k in range(RPT // ZR):
        pltpu.sync_copy(z_hbm, acc.at[pl.ds(s * RPT + k * ZR, ZR)])
    plsc.subcore_barrier()

    def outer(g, carry):
        pltpu.sync_copy(col_hbm.at[wid, g], idx_v)

        def step(i, c2):
            pltpu.sync_copy(ones_v, acc.at[idx_v.at[i]], add=True)
            return c2

        return lax.fori_loop(0, IB, step, carry)

    lax.fori_loop(0, K1_G, outer, 0)
    plsc.subcore_barrier()
    sl = pl.ds(s * RPT, RPT)
    pltpu.sync_copy(acc.at[sl], dp_hbm.at[c, sl])


_deg_call = pl.kernel(
    _deg_body,
    out_type=jax.ShapeDtypeStruct((NC, NP, D), jnp.float32),
    mesh=plsc.VectorSubcoreMesh(core_axis_name="c", subcore_axis_name="s"),
    scratch_types=[
        pltpu.VMEM((CH, D), jnp.float32),
        pltpu.VMEM((IB, CH), jnp.int32),
        pltpu.VMEM_SHARED((NP, D), jnp.float32),
    ],
)


def _agg_body(row_hbm, col_hbm, u_hbm, v_hbm, z_hbm,
              U_hbm, V_hbm, idxr, idxc, gb0, gb1, acc, sm0, sm1):
    c = lax.axis_index("c")
    s = lax.axis_index("s")
    gbufs = [gb0, gb1]
    sems = [sm0, sm1]
    for k in range(RPT // ZR):
        pltpu.sync_copy(z_hbm, acc.at[pl.ds(s * RPT + k * ZR, ZR)])
    plsc.subcore_barrier()

    def outer(g, carry):
        pltpu.sync_copy(row_hbm.at[s, g], idxr)
        pltpu.sync_copy(col_hbm.at[s, g], idxc)

        def group(j, c2):
            for k in range(GB):
                i = j * GB + k

                @pl.when(c == 0)
                def _(k=k, i=i):
                    pltpu.async_copy(u_hbm.at[idxc.at[i]], gbufs[k], sems[k])

                @pl.when(c == 1)
                def _(k=k, i=i):
                    pltpu.async_copy(v_hbm.at[idxc.at[i]], gbufs[k], sems[k])

            for k in range(GB):
                i = j * GB + k

                @pl.when(c == 0)
                def _(k=k, i=i):
                    pltpu.make_async_copy(
                        u_hbm.at[idxc.at[i]], gbufs[k], sems[k]).wait()

                @pl.when(c == 1)
                def _(k=k, i=i):
                    pltpu.make_async_copy(
                        v_hbm.at[idxc.at[i]], gbufs[k], sems[k]).wait()

                pltpu.sync_copy(gbufs[k], acc.at[idxr.at[i]], add=True)
            return c2

        return lax.fori_loop(0, IBA // GB, group, carry)

    lax.fori_loop(0, K3_G, outer, 0)
    plsc.subcore_barrier()
    sl = pl.ds(s * RPT, RPT)

    @pl.when(c == 0)
    def _():
        pltpu.sync_copy(acc.at[sl], U_hbm.at[sl])

    @pl.when(c == 1)
    def _():
        pltpu.sync_copy(acc.at[sl], V_hbm.at[sl])


_agg_call = pl.kernel(
    _agg_body,
    out_type=(
        jax.ShapeDtypeStruct((NP, D), jnp.float32),
        jax.ShapeDtypeStruct((NP, D), jnp.float32),
    ),
    mesh=plsc.VectorSubcoreMesh(core_axis_name="c", subcore_axis_name="s"),
    scratch_types=[
        pltpu.VMEM((IBA, CH), jnp.int32),
        pltpu.VMEM((IBA, CH), jnp.int32),
        pltpu.VMEM((CH, D), jnp.float32),
        pltpu.VMEM((CH, D), jnp.float32),
        pltpu.VMEM_SHARED((NP, D), jnp.float32),
        pltpu.SemaphoreType.DMA,
        pltpu.SemaphoreType.DMA,
    ],
)


def _mx_body(x_ref, o_ref):
    o_ref[0, 0] = jnp.max(x_ref[...])


def _mx_call(x):
    return pl.pallas_call(
        _mx_body,
        out_shape=jax.ShapeDtypeStruct((1, 1), jnp.float32),
        out_specs=pl.BlockSpec(memory_space=pltpu.SMEM),
    )(x)


def _tab_body(mx_ref, p_ref, x_ref, dp_ref, u_ref, v_ref):
    pp = jax.nn.sigmoid(p_ref[0, 0]) * 2.0
    m = pp * mx_ref[0, 0]
    dp = dp_ref[...]
    deg = dp[0, :, 0] + dp[1, :, 0]
    dis = jnp.where(deg > 0, lax.rsqrt(deg), 0.0)
    xb = x_ref[...]
    u = jnp.exp(pp * xb - m) * dis[:, None]
    u_ref[...] = u
    v_ref[...] = u * xb


def _tab_call(mx, p2, x, dp):
    return pl.pallas_call(
        _tab_body,
        grid=(N // RB,),
        in_specs=[
            pl.BlockSpec((1, 1), lambda i: (0, 0), memory_space=pltpu.SMEM),
            pl.BlockSpec((1, 1), lambda i: (0, 0), memory_space=pltpu.SMEM),
            pl.BlockSpec((RB, D), lambda i: (i, 0)),
            pl.BlockSpec((NC, RB, D), lambda i: (0, i, 0)),
        ],
        out_specs=[
            pl.BlockSpec((RB, D), lambda i: (i, 0)),
            pl.BlockSpec((RB, D), lambda i: (i, 0)),
        ],
        out_shape=[
            jax.ShapeDtypeStruct((N, D), jnp.float32),
            jax.ShapeDtypeStruct((N, D), jnp.float32),
        ],
    )(mx, p2, x, dp)


def _out_body(e_ref, U_ref, V_ref, x_ref, dp_ref, o_ref):
    dp = dp_ref[...]
    deg = dp[0, :, 0] + dp[1, :, 0]
    dis = jnp.where(deg > 0, lax.rsqrt(deg), 0.0)[:, None]
    xb = x_ref[...]
    agg = (dis * V_ref[...]) / (dis * U_ref[...] + 1e-6)
    o_ref[...] = agg + (1.0 + e_ref[0, 0]) * xb


def _out_call(e2, U, V, x, dp):
    return pl.pallas_call(
        _out_body,
        grid=(N // RB,),
        in_specs=[
            pl.BlockSpec((1, 1), lambda i: (0, 0), memory_space=pltpu.SMEM),
            pl.BlockSpec((RB, D), lambda i: (i, 0)),
            pl.BlockSpec((RB, D), lambda i: (i, 0)),
            pl.BlockSpec((RB, D), lambda i: (i, 0)),
            pl.BlockSpec((NC, RB, D), lambda i: (0, i, 0)),
        ],
        out_specs=pl.BlockSpec((RB, D), lambda i: (i, 0)),
        out_shape=jax.ShapeDtypeStruct((N, D), jnp.float32),
    )(e2, U, V, x, dp)


def kernel(x, edge_index, eps, p):
    row = edge_index[0]
    col = edge_index[1]
    row3 = row.reshape(NS, K3_G, IBA, CH)
    col3 = col.reshape(NS, K3_G, IBA, CH)
    col3d = col.reshape(NC * NS, K1_G, IB, CH)
    onesD = jnp.ones((CH, D), jnp.float32)
    zD = jnp.zeros((ZR, D), jnp.float32)

    dp = _deg_call(col3d, onesD, zD)
    mx = _mx_call(x)
    u, v = _tab_call(mx, p.reshape(1, 1), x, dp)
    U, V = _agg_call(row3, col3, u, v, zD)
    return _out_call(eps.reshape(1, 1), U, V, x, dp)


# speedup vs baseline: 17.0773x; 1.0394x over previous
import jax
import jax.numpy as jnp
from jax import lax
from jax.experimental import pallas as pl
from jax.experimental.pallas import tpu as pltpu
from jax.experimental.pallas import tpu_sc as plsc

N = 10000
E = 320000
D = 128
CH = 80
NC = 2
NS = 16
NP = 10240
RPT = NP // NS
ZR = 128
IB = 25
IBA = 50
CHA = 40
GB = 5
K1_ROWS = E // CH // (NC * NS)
K3_ROWS = E // CHA // NS
K1_G = K1_ROWS // IB
K3_G = K3_ROWS // IBA
RB = 1000


def _deg_body(col_hbm, ones_hbm, z_hbm, dp_hbm, ones_v, idx_v, acc):
    c = lax.axis_index("c")
    s = lax.axis_index("s")
    wid = c * NS + s
    pltpu.sync_copy(ones_hbm, ones_v)
    for k in range(RPT // ZR):
        pltpu.sync_copy(z_hbm, acc.at[pl.ds(s * RPT + k * ZR, ZR)])
    plsc.subcore_barrier()

    def outer(g, carry):
        pltpu.sync_copy(col_hbm.at[wid, g], idx_v)

        def step(i, c2):
            pltpu.sync_copy(ones_v, acc.at[idx_v.at[i]], add=True)
            return c2

        return lax.fori_loop(0, IB, step, carry)

    lax.fori_loop(0, K1_G, outer, 0)
    plsc.subcore_barrier()
    sl = pl.ds(s * RPT, RPT)
    pltpu.sync_copy(acc.at[sl], dp_hbm.at[c, sl])


_deg_call = pl.kernel(
    _deg_body,
    out_type=jax.ShapeDtypeStruct((NC, NP, D), jnp.float32),
    mesh=plsc.VectorSubcoreMesh(core_axis_name="c", subcore_axis_name="s"),
    scratch_types=[
        pltpu.VMEM((CH, D), jnp.float32),
        pltpu.VMEM((IB, CH), jnp.int32),
        pltpu.VMEM_SHARED((NP, D), jnp.float32),
    ],
)


def _agg_body(row_hbm, col_hbm, u_hbm, v_hbm, z_hbm,
              U_hbm, V_hbm, idxr, idxc, gb0, gb1, gb2, gb3, gb4, acc,
              sg0, sg1, sg2, sg3, sg4, ss0, ss1, ss2, ss3, ss4):
    c = lax.axis_index("c")
    s = lax.axis_index("s")
    gbufs = [gb0, gb1, gb2, gb3, gb4]
    gsems = [sg0, sg1, sg2, sg3, sg4]
    ssems = [ss0, ss1, ss2, ss3, ss4]
    for k in range(RPT // ZR):
        pltpu.sync_copy(z_hbm, acc.at[pl.ds(s * RPT + k * ZR, ZR)])
    plsc.subcore_barrier()

    def outer(g, carry):
        pltpu.sync_copy(row_hbm.at[s, g], idxr)
        pltpu.sync_copy(col_hbm.at[s, g], idxc)

        def group(j, c2):
            for k in range(GB):
                i = j * GB + k

                @pl.when(c == 0)
                def _(k=k, i=i):
                    pltpu.async_copy(u_hbm.at[idxc.at[i]], gbufs[k], gsems[k])

                @pl.when(c == 1)
                def _(k=k, i=i):
                    pltpu.async_copy(v_hbm.at[idxc.at[i]], gbufs[k], gsems[k])

            for k in range(GB):
                i = j * GB + k

                @pl.when(c == 0)
                def _(k=k, i=i):
                    pltpu.make_async_copy(
                        u_hbm.at[idxc.at[i]], gbufs[k], gsems[k]).wait()

                @pl.when(c == 1)
                def _(k=k, i=i):
                    pltpu.make_async_copy(
                        v_hbm.at[idxc.at[i]], gbufs[k], gsems[k]).wait()

                pltpu.async_copy(gbufs[k], acc.at[idxr.at[i]], ssems[k],
                                 add=True)

            for k in range(GB):
                i = j * GB + k
                pltpu.make_async_copy(
                    gbufs[k], acc.at[idxr.at[i]], ssems[k]).wait()
            return c2

        return lax.fori_loop(0, IBA // GB, group, carry)

    lax.fori_loop(0, K3_G, outer, 0)
    plsc.subcore_barrier()
    sl = pl.ds(s * RPT, RPT)

    @pl.when(c == 0)
    def _():
        pltpu.sync_copy(acc.at[sl], U_hbm.at[sl])

    @pl.when(c == 1)
    def _():
        pltpu.sync_copy(acc.at[sl], V_hbm.at[sl])


_agg_call = pl.kernel(
    _agg_body,
    out_type=(
        jax.ShapeDtypeStruct((NP, D), jnp.float32),
        jax.ShapeDtypeStruct((NP, D), jnp.float32),
    ),
    mesh=plsc.VectorSubcoreMesh(core_axis_name="c", subcore_axis_name="s"),
    scratch_types=(
        [pltpu.VMEM((IBA, CHA), jnp.int32)] * 2
        + [pltpu.VMEM((CHA, D), jnp.float32)] * GB
        + [pltpu.VMEM_SHARED((NP, D), jnp.float32)]
        + [pltpu.SemaphoreType.DMA] * (2 * GB)
    ),
)


def _mx_body(x_ref, o_ref):
    o_ref[0, 0] = jnp.max(x_ref[...])


def _mx_call(x):
    return pl.pallas_call(
        _mx_body,
        out_shape=jax.ShapeDtypeStruct((1, 1), jnp.float32),
        out_specs=pl.BlockSpec(memory_space=pltpu.SMEM),
    )(x)


def _tab_body(mx_ref, p_ref, x_ref, dp_ref, u_ref, v_ref):
    pp = jax.nn.sigmoid(p_ref[0, 0]) * 2.0
    m = pp * mx_ref[0, 0]
    dp = dp_ref[...]
    deg = dp[0, :, 0] + dp[1, :, 0]
    dis = jnp.where(deg > 0, lax.rsqrt(deg), 0.0)
    xb = x_ref[...]
    u = jnp.exp(pp * xb - m) * dis[:, None]
    u_ref[...] = u
    v_ref[...] = u * xb


def _tab_call(mx, p2, x, dp):
    return pl.pallas_call(
        _tab_body,
        grid=(N // RB,),
        in_specs=[
            pl.BlockSpec((1, 1), lambda i: (0, 0), memory_space=pltpu.SMEM),
            pl.BlockSpec((1, 1), lambda i: (0, 0), memory_space=pltpu.SMEM),
            pl.BlockSpec((RB, D), lambda i: (i, 0)),
            pl.BlockSpec((NC, RB, D), lambda i: (0, i, 0)),
        ],
        out_specs=[
            pl.BlockSpec((RB, D), lambda i: (i, 0)),
            pl.BlockSpec((RB, D), lambda i: (i, 0)),
        ],
        out_shape=[
            jax.ShapeDtypeStruct((N, D), jnp.float32),
            jax.ShapeDtypeStruct((N, D), jnp.float32),
        ],
    )(mx, p2, x, dp)


def _out_body(e_ref, U_ref, V_ref, x_ref, dp_ref, o_ref):
    dp = dp_ref[...]
    deg = dp[0, :, 0] + dp[1, :, 0]
    dis = jnp.where(deg > 0, lax.rsqrt(deg), 0.0)[:, None]
    xb = x_ref[...]
    agg = (dis * V_ref[...]) / (dis * U_ref[...] + 1e-6)
    o_ref[...] = agg + (1.0 + e_ref[0, 0]) * xb


def _out_call(e2, U, V, x, dp):
    return pl.pallas_call(
        _out_body,
        grid=(N // RB,),
        in_specs=[
            pl.BlockSpec((1, 1), lambda i: (0, 0), memory_space=pltpu.SMEM),
            pl.BlockSpec((RB, D), lambda i: (i, 0)),
            pl.BlockSpec((RB, D), lambda i: (i, 0)),
            pl.BlockSpec((RB, D), lambda i: (i, 0)),
            pl.BlockSpec((NC, RB, D), lambda i: (0, i, 0)),
        ],
        out_specs=pl.BlockSpec((RB, D), lambda i: (i, 0)),
        out_shape=jax.ShapeDtypeStruct((N, D), jnp.float32),
    )(e2, U, V, x, dp)


def kernel(x, edge_index, eps, p):
    row = edge_index[0]
    col = edge_index[1]
    row3 = row.reshape(NS, K3_G, IBA, CHA)
    col3 = col.reshape(NS, K3_G, IBA, CHA)
    col3d = col.reshape(NC * NS, K1_G, IB, CH)
    onesD = jnp.ones((CH, D), jnp.float32)
    zD = jnp.zeros((ZR, D), jnp.float32)

    dp = _deg_call(col3d, onesD, zD)
    mx = _mx_call(x)
    u, v = _tab_call(mx, p.reshape(1, 1), x, dp)
    U, V = _agg_call(row3, col3, u, v, zD)
    return _out_call(eps.reshape(1, 1), U, V, x, dp)


# speedup vs baseline: 19.7224x; 1.1549x over previous
import jax
import jax.numpy as jnp
from jax import lax
from jax.experimental import pallas as pl
from jax.experimental.pallas import tpu as pltpu
from jax.experimental.pallas import tpu_sc as plsc

N = 10000
E = 320000
D = 128
CH = 80
NC = 2
NS = 16
NP = 10240
RPT = NP // NS
ZR = 128
IB = 25
IBA = 50
CHA = 40
GB = 5
K1_ROWS = E // CH // (NC * NS)
K3_ROWS = E // CHA // NS
K1_G = K1_ROWS // IB
K3_G = K3_ROWS // IBA
RB = 1000


def _deg_body(col_hbm, ones_hbm, z_hbm, dp_hbm, ones_v, idx_v, acc):
    c = lax.axis_index("c")
    s = lax.axis_index("s")
    wid = c * NS + s
    pltpu.sync_copy(ones_hbm, ones_v)
    for k in range(RPT // ZR):
        pltpu.sync_copy(z_hbm, acc.at[pl.ds(s * RPT + k * ZR, ZR)])
    plsc.subcore_barrier()

    def outer(g, carry):
        pltpu.sync_copy(col_hbm.at[wid, g], idx_v)

        def step(i, c2):
            pltpu.sync_copy(ones_v, acc.at[idx_v.at[i]], add=True)
            return c2

        return lax.fori_loop(0, IB, step, carry)

    lax.fori_loop(0, K1_G, outer, 0)
    plsc.subcore_barrier()
    sl = pl.ds(s * RPT, RPT)
    pltpu.sync_copy(acc.at[sl], dp_hbm.at[c, sl])


_deg_call = pl.kernel(
    _deg_body,
    out_type=jax.ShapeDtypeStruct((NC, NP, D), jnp.float32),
    mesh=plsc.VectorSubcoreMesh(core_axis_name="c", subcore_axis_name="s"),
    scratch_types=[
        pltpu.VMEM((CH, D), jnp.float32),
        pltpu.VMEM((IB, CH), jnp.int32),
        pltpu.VMEM_SHARED((NP, D), jnp.float32),
    ],
)


def _agg_body(row_hbm, col_hbm, u_hbm, v_hbm, z_hbm,
              U_hbm, V_hbm, idxr, idxc, gb0, gb1, gb2, gb3, gb4, acc,
              sg0, sg1, sg2, sg3, sg4, ss0, ss1, ss2, ss3, ss4):
    c = lax.axis_index("c")
    s = lax.axis_index("s")
    gbufs = [gb0, gb1, gb2, gb3, gb4]
    gsems = [sg0, sg1, sg2, sg3, sg4]
    ssems = [ss0, ss1, ss2, ss3, ss4]
    for k in range(RPT // ZR):
        pltpu.sync_copy(z_hbm, acc.at[pl.ds(s * RPT + k * ZR, ZR)])
    plsc.subcore_barrier()

    def outer(g, carry):
        pltpu.sync_copy(row_hbm.at[s, g], idxr)
        pltpu.sync_copy(col_hbm.at[s, g], idxc)

        def group(j, c2):
            for k in range(GB):
                i = j * GB + k

                @pl.when(j > 0)
                def _(k=k, i=i):
                    pltpu.make_async_copy(
                        gbufs[k], acc.at[idxr.at[i]], ssems[k]).wait()

                @pl.when(c == 0)
                def _(k=k, i=i):
                    pltpu.async_copy(u_hbm.at[idxc.at[i]], gbufs[k], gsems[k])

                @pl.when(c == 1)
                def _(k=k, i=i):
                    pltpu.async_copy(v_hbm.at[idxc.at[i]], gbufs[k], gsems[k])

            for k in range(GB):
                i = j * GB + k

                @pl.when(c == 0)
                def _(k=k, i=i):
                    pltpu.make_async_copy(
                        u_hbm.at[idxc.at[i]], gbufs[k], gsems[k]).wait()

                @pl.when(c == 1)
                def _(k=k, i=i):
                    pltpu.make_async_copy(
                        v_hbm.at[idxc.at[i]], gbufs[k], gsems[k]).wait()

                pltpu.async_copy(gbufs[k], acc.at[idxr.at[i]], ssems[k],
                                 add=True)

            return c2

        r = lax.fori_loop(0, IBA // GB, group, carry)
        for k in range(GB):
            i = IBA - GB + k
            pltpu.make_async_copy(
                gbufs[k], acc.at[idxr.at[i]], ssems[k]).wait()
        return r

    lax.fori_loop(0, K3_G, outer, 0)
    plsc.subcore_barrier()
    sl = pl.ds(s * RPT, RPT)

    @pl.when(c == 0)
    def _():
        pltpu.sync_copy(acc.at[sl], U_hbm.at[sl])

    @pl.when(c == 1)
    def _():
        pltpu.sync_copy(acc.at[sl], V_hbm.at[sl])


_agg_call = pl.kernel(
    _agg_body,
    out_type=(
        jax.ShapeDtypeStruct((NP, D), jnp.float32),
        jax.ShapeDtypeStruct((NP, D), jnp.float32),
    ),
    mesh=plsc.VectorSubcoreMesh(core_axis_name="c", subcore_axis_name="s"),
    scratch_types=(
        [pltpu.VMEM((IBA, CHA), jnp.int32)] * 2
        + [pltpu.VMEM((CHA, D), jnp.float32)] * GB
        + [pltpu.VMEM_SHARED((NP, D), jnp.float32)]
        + [pltpu.SemaphoreType.DMA] * (2 * GB)
    ),
)


def _mx_body(x_ref, o_ref):
    o_ref[0, 0] = jnp.max(x_ref[...])


def _mx_call(x):
    return pl.pallas_call(
        _mx_body,
        out_shape=jax.ShapeDtypeStruct((1, 1), jnp.float32),
        out_specs=pl.BlockSpec(memory_space=pltpu.SMEM),
    )(x)


def _tab_body(mx_ref, p_ref, x_ref, dp_ref, u_ref, v_ref):
    pp = jax.nn.sigmoid(p_ref[0, 0]) * 2.0
    m = pp * mx_ref[0, 0]
    dp = dp_ref[...]
    deg = dp[0, :, 0] + dp[1, :, 0]
    dis = jnp.where(deg > 0, lax.rsqrt(deg), 0.0)
    xb = x_ref[...]
    u = jnp.exp(pp * xb - m) * dis[:, None]
    u_ref[...] = u
    v_ref[...] = u * xb


def _tab_call(mx, p2, x, dp):
    return pl.pallas_call(
        _tab_body,
        grid=(N // RB,),
        in_specs=[
            pl.BlockSpec((1, 1), lambda i: (0, 0), memory_space=pltpu.SMEM),
            pl.BlockSpec((1, 1), lambda i: (0, 0), memory_space=pltpu.SMEM),
            pl.BlockSpec((RB, D), lambda i: (i, 0)),
            pl.BlockSpec((NC, RB, D), lambda i: (0, i, 0)),
        ],
        out_specs=[
            pl.BlockSpec((RB, D), lambda i: (i, 0)),
            pl.BlockSpec((RB, D), lambda i: (i, 0)),
        ],
        out_shape=[
            jax.ShapeDtypeStruct((N, D), jnp.float32),
            jax.ShapeDtypeStruct((N, D), jnp.float32),
        ],
    )(mx, p2, x, dp)


def _out_body(e_ref, U_ref, V_ref, x_ref, dp_ref, o_ref):
    dp = dp_ref[...]
    deg = dp[0, :, 0] + dp[1, :, 0]
    dis = jnp.where(deg > 0, lax.rsqrt(deg), 0.0)[:, None]
    xb = x_ref[...]
    agg = (dis * V_ref[...]) / (dis * U_ref[...] + 1e-6)
    o_ref[...] = agg + (1.0 + e_ref[0, 0]) * xb


def _out_call(e2, U, V, x, dp):
    return pl.pallas_call(
        _out_body,
        grid=(N // RB,),
        in_specs=[
            pl.BlockSpec((1, 1), lambda i: (0, 0), memory_space=pltpu.SMEM),
            pl.BlockSpec((RB, D), lambda i: (i, 0)),
            pl.BlockSpec((RB, D), lambda i: (i, 0)),
            pl.BlockSpec((RB, D), lambda i: (i, 0)),
            pl.BlockSpec((NC, RB, D), lambda i: (0, i, 0)),
        ],
        out_specs=pl.BlockSpec((RB, D), lambda i: (i, 0)),
        out_shape=jax.ShapeDtypeStruct((N, D), jnp.float32),
    )(e2, U, V, x, dp)


def kernel(x, edge_index, eps, p):
    row = edge_index[0]
    col = edge_index[1]
    row3 = row.reshape(NS, K3_G, IBA, CHA)
    col3 = col.reshape(NS, K3_G, IBA, CHA)
    col3d = col.reshape(NC * NS, K1_G, IB, CH)
    onesD = jnp.ones((CH, D), jnp.float32)
    zD = jnp.zeros((ZR, D), jnp.float32)

    dp = _deg_call(col3d, onesD, zD)
    mx = _mx_call(x)
    u, v = _tab_call(mx, p.reshape(1, 1), x, dp)
    U, V = _agg_call(row3, col3, u, v, zD)
    return _out_call(eps.reshape(1, 1), U, V, x, dp)


# speedup vs baseline: 19.8012x; 1.0040x over previous
import jax
import jax.numpy as jnp
from jax import lax
from jax.experimental import pallas as pl
from jax.experimental.pallas import tpu as pltpu
from jax.experimental.pallas import tpu_sc as plsc

N = 10000
E = 320000
D = 128
CH = 80
NC = 2
NS = 16
NP = 10240
RPT = NP // NS
ZR = 128
IB = 25
IBA = 50
CHA = 40
GB = 5
K1_ROWS = E // CH // (NC * NS)
K3_ROWS = E // CHA // NS
K1_G = K1_ROWS // IB
K3_G = K3_ROWS // IBA
RB = 1000
DW = 128


def _deg_body(col_hbm, ones_hbm, z_hbm, dp_hbm, ones_v, idx_v, acc, sem):
    c = lax.axis_index("c")
    s = lax.axis_index("s")
    wid = c * NS + s
    pltpu.sync_copy(ones_hbm, ones_v)
    for k in range(RPT // ZR):
        pltpu.sync_copy(z_hbm, acc.at[pl.ds(s * RPT + k * ZR, ZR)])
    plsc.subcore_barrier()

    def outer(g, carry):
        pltpu.sync_copy(col_hbm.at[wid, g], idx_v)

        def fire(i, c2):
            pltpu.async_copy(ones_v, acc.at[idx_v.at[i]], sem, add=True)
            return c2

        def drain(i, c2):
            pltpu.make_async_copy(ones_v, acc.at[idx_v.at[i]], sem).wait()
            return c2

        r = lax.fori_loop(0, IB, fire, carry)
        return lax.fori_loop(0, IB, drain, r)

    lax.fori_loop(0, K1_G, outer, 0)
    plsc.subcore_barrier()
    sl = pl.ds(s * RPT, RPT)
    pltpu.sync_copy(acc.at[sl], dp_hbm.at[c, sl])


_deg_call = pl.kernel(
    _deg_body,
    out_type=jax.ShapeDtypeStruct((NC, NP, DW), jnp.float32),
    mesh=plsc.VectorSubcoreMesh(core_axis_name="c", subcore_axis_name="s"),
    scratch_types=[
        pltpu.VMEM((CH, DW), jnp.float32),
        pltpu.VMEM((IB, CH), jnp.int32),
        pltpu.VMEM_SHARED((NP, DW), jnp.float32),
        pltpu.SemaphoreType.DMA,
    ],
)


def _agg_body(row_hbm, col_hbm, u_hbm, v_hbm, z_hbm,
              U_hbm, V_hbm, idxr, idxc, gb0, gb1, gb2, gb3, gb4, acc,
              sg0, sg1, sg2, sg3, sg4, ss0, ss1, ss2, ss3, ss4):
    c = lax.axis_index("c")
    s = lax.axis_index("s")
    gbufs = [gb0, gb1, gb2, gb3, gb4]
    gsems = [sg0, sg1, sg2, sg3, sg4]
    ssems = [ss0, ss1, ss2, ss3, ss4]
    for k in range(RPT // ZR):
        pltpu.sync_copy(z_hbm, acc.at[pl.ds(s * RPT + k * ZR, ZR)])
    plsc.subcore_barrier()

    def outer(g, carry):
        pltpu.sync_copy(row_hbm.at[s, g], idxr)
        pltpu.sync_copy(col_hbm.at[s, g], idxc)

        def group(j, c2):
            for k in range(GB):
                i = j * GB + k

                @pl.when(j > 0)
                def _(k=k, i=i):
                    pltpu.make_async_copy(
                        gbufs[k], acc.at[idxr.at[i]], ssems[k]).wait()

                @pl.when(c == 0)
                def _(k=k, i=i):
                    pltpu.async_copy(u_hbm.at[idxc.at[i]], gbufs[k], gsems[k])

                @pl.when(c == 1)
                def _(k=k, i=i):
                    pltpu.async_copy(v_hbm.at[idxc.at[i]], gbufs[k], gsems[k])

            for k in range(GB):
                i = j * GB + k

                @pl.when(c == 0)
                def _(k=k, i=i):
                    pltpu.make_async_copy(
                        u_hbm.at[idxc.at[i]], gbufs[k], gsems[k]).wait()

                @pl.when(c == 1)
                def _(k=k, i=i):
                    pltpu.make_async_copy(
                        v_hbm.at[idxc.at[i]], gbufs[k], gsems[k]).wait()

                pltpu.async_copy(gbufs[k], acc.at[idxr.at[i]], ssems[k],
                                 add=True)

            return c2

        r = lax.fori_loop(0, IBA // GB, group, carry)
        for k in range(GB):
            i = IBA - GB + k
            pltpu.make_async_copy(
                gbufs[k], acc.at[idxr.at[i]], ssems[k]).wait()
        return r

    lax.fori_loop(0, K3_G, outer, 0)
    plsc.subcore_barrier()
    sl = pl.ds(s * RPT, RPT)

    @pl.when(c == 0)
    def _():
        pltpu.sync_copy(acc.at[sl], U_hbm.at[sl])

    @pl.when(c == 1)
    def _():
        pltpu.sync_copy(acc.at[sl], V_hbm.at[sl])


_agg_call = pl.kernel(
    _agg_body,
    out_type=(
        jax.ShapeDtypeStruct((NP, D), jnp.float32),
        jax.ShapeDtypeStruct((NP, D), jnp.float32),
    ),
    mesh=plsc.VectorSubcoreMesh(core_axis_name="c", subcore_axis_name="s"),
    scratch_types=(
        [pltpu.VMEM((IBA, CHA), jnp.int32)] * 2
        + [pltpu.VMEM((CHA, D), jnp.float32)] * GB
        + [pltpu.VMEM_SHARED((NP, D), jnp.float32)]
        + [pltpu.SemaphoreType.DMA] * (2 * GB)
    ),
)


def _mx_body(x_ref, o_ref):
    o_ref[0, 0] = jnp.max(x_ref[...])


def _mx_call(x):
    return pl.pallas_call(
        _mx_body,
        out_shape=jax.ShapeDtypeStruct((1, 1), jnp.float32),
        out_specs=pl.BlockSpec(memory_space=pltpu.SMEM),
    )(x)


def _tab_body(mx_ref, p_ref, x_ref, dp_ref, u_ref, v_ref):
    pp = jax.nn.sigmoid(p_ref[0, 0]) * 2.0
    m = pp * mx_ref[0, 0]
    dp = dp_ref[...]
    deg = dp[0, :, 0] + dp[1, :, 0]
    dis = jnp.where(deg > 0, lax.rsqrt(deg), 0.0)
    xb = x_ref[...]
    u = jnp.exp(pp * xb - m) * dis[:, None]
    u_ref[...] = u
    v_ref[...] = u * xb


def _tab_call(mx, p2, x, dp):
    return pl.pallas_call(
        _tab_body,
        grid=(N // RB,),
        in_specs=[
            pl.BlockSpec((1, 1), lambda i: (0, 0), memory_space=pltpu.SMEM),
            pl.BlockSpec((1, 1), lambda i: (0, 0), memory_space=pltpu.SMEM),
            pl.BlockSpec((RB, D), lambda i: (i, 0)),
            pl.BlockSpec((NC, RB, DW), lambda i: (0, i, 0)),
        ],
        out_specs=[
            pl.BlockSpec((RB, D), lambda i: (i, 0)),
            pl.BlockSpec((RB, D), lambda i: (i, 0)),
        ],
        out_shape=[
            jax.ShapeDtypeStruct((N, D), jnp.float32),
            jax.ShapeDtypeStruct((N, D), jnp.float32),
        ],
    )(mx, p2, x, dp)


def _out_body(e_ref, U_ref, V_ref, x_ref, dp_ref, o_ref):
    dp = dp_ref[...]
    deg = dp[0, :, 0] + dp[1, :, 0]
    dis = jnp.where(deg > 0, lax.rsqrt(deg), 0.0)[:, None]
    xb = x_ref[...]
    agg = (dis * V_ref[...]) / (dis * U_ref[...] + 1e-6)
    o_ref[...] = agg + (1.0 + e_ref[0, 0]) * xb


def _out_call(e2, U, V, x, dp):
    return pl.pallas_call(
        _out_body,
        grid=(N // RB,),
        in_specs=[
            pl.BlockSpec((1, 1), lambda i: (0, 0), memory_space=pltpu.SMEM),
            pl.BlockSpec((RB, D), lambda i: (i, 0)),
            pl.BlockSpec((RB, D), lambda i: (i, 0)),
            pl.BlockSpec((RB, D), lambda i: (i, 0)),
            pl.BlockSpec((NC, RB, DW), lambda i: (0, i, 0)),
        ],
        out_specs=pl.BlockSpec((RB, D), lambda i: (i, 0)),
        out_shape=jax.ShapeDtypeStruct((N, D), jnp.float32),
    )(e2, U, V, x, dp)


def kernel(x, edge_index, eps, p):
    row = edge_index[0]
    col = edge_index[1]
    row3 = row.reshape(NS, K3_G, IBA, CHA)
    col3 = col.reshape(NS, K3_G, IBA, CHA)
    col3d = col.reshape(NC * NS, K1_G, IB, CH)
    onesW = jnp.ones((CH, DW), jnp.float32)
    zW = jnp.zeros((ZR, DW), jnp.float32)
    zD = jnp.zeros((ZR, D), jnp.float32)

    dp = _deg_call(col3d, onesW, zW)
    mx = _mx_call(x)
    u, v = _tab_call(mx, p.reshape(1, 1), x, dp)
    U, V = _agg_call(row3, col3, u, v, zD)
    return _out_call(eps.reshape(1, 1), U, V, x, dp)
